# baseline (speedup 1.0000x reference)
import jax
import jax.numpy as jnp
from jax.experimental import pallas as pl
from jax.experimental.pallas import tpu as pltpu


def _make_outer(n_chunks, tc, S, A):
    def _outer(x_hbm, w1_ref, b1_ref, w2_ref, b2_ref, out_hbm):
        def _inner(x_blk, o_blk):
            xb = x_blk[...].astype(jnp.bfloat16)
            w1b = w1_ref[...].astype(jnp.bfloat16)
            hid = jnp.dot(xb, w1b, preferred_element_type=jnp.float32)
            hid = jnp.maximum(hid + b1_ref[...], 0.0).astype(jnp.bfloat16)
            w2b = w2_ref[...].astype(jnp.bfloat16)
            out = jnp.dot(hid, w2b, preferred_element_type=jnp.float32)
            o_blk[...] = out + b2_ref[...]

        pipeline = pltpu.emit_pipeline(
            _inner,
            grid=(n_chunks,),
            in_specs=[pl.BlockSpec((tc, S), lambda i: (i, 0))],
            out_specs=[pl.BlockSpec((tc, A), lambda i: (i, 0))],
        )
        pipeline(x_hbm, out_hbm)

    return _outer


def _emitter_fallback(x, w1, b1, w2, b2):
    def _mlp_body(x_ref, w1_ref, b1_ref, w2_ref, b2_ref, out_ref):
        xb = x_ref[...].astype(jnp.bfloat16)
        w1b = w1_ref[...].astype(jnp.bfloat16)
        hid = jnp.dot(xb, w1b, preferred_element_type=jnp.float32)
        hid = jnp.maximum(hid + b1_ref[...], 0.0).astype(jnp.bfloat16)
        w2b = w2_ref[...].astype(jnp.bfloat16)
        out = jnp.dot(hid, w2b, preferred_element_type=jnp.float32)
        out_ref[...] = out + b2_ref[...]

    B, S = x.shape
    H = w1.shape[1]
    A = w2.shape[1]
    TB = min(8192, B)
    nb = pl.cdiv(B, TB)
    return pl.pallas_call(
        _mlp_body,
        out_shape=jax.ShapeDtypeStruct((B, A), x.dtype),
        grid=(nb,),
        in_specs=[
            pl.BlockSpec((TB, S), lambda i: (i, 0)),
            pl.BlockSpec((S, H), lambda i: (0, 0)),
            pl.BlockSpec((1, H), lambda i: (0, 0)),
            pl.BlockSpec((H, A), lambda i: (0, 0)),
            pl.BlockSpec((1, A), lambda i: (0, 0)),
        ],
        out_specs=pl.BlockSpec((TB, A), lambda i: (i, 0)),
        compiler_params=pltpu.CompilerParams(
            dimension_semantics=("parallel",),
        ),
    )(x, w1, b1, w2, b2)


@jax.jit
def kernel(x, w1, b1, w2, b2):
    B, S = x.shape
    H = w1.shape[1]
    A = w2.shape[1]

    TC = 2048
    if B % TC != 0 or B // TC < 2:
        return _emitter_fallback(x, w1, b1, w2, b2)
    n_chunks = B // TC

    return pl.pallas_call(
        _make_outer(n_chunks, TC, S, A),
        out_shape=jax.ShapeDtypeStruct((B, A), x.dtype),
        in_specs=[
            pl.BlockSpec(memory_space=pltpu.HBM),
            pl.BlockSpec(memory_space=pltpu.VMEM),
            pl.BlockSpec(memory_space=pltpu.VMEM),
            pl.BlockSpec(memory_space=pltpu.VMEM),
            pl.BlockSpec(memory_space=pltpu.VMEM),
        ],
        out_specs=pl.BlockSpec(memory_space=pltpu.HBM),
    )(x, w1, b1, w2, b2)


# speedup vs baseline: 1.1978x; 1.1978x over previous
import jax
import jax.numpy as jnp
from jax.experimental import pallas as pl
from jax.experimental.pallas import tpu as pltpu


def _mlp_body(x_ref, w1_ref, b1_ref, w2_ref, b2_ref, out_ref):
    x = x_ref[...].astype(jnp.bfloat16)
    w1 = w1_ref[...].astype(jnp.bfloat16)
    hid = jnp.dot(x, w1, preferred_element_type=jnp.float32)
    hid = jnp.maximum(hid + b1_ref[...], 0.0).astype(jnp.bfloat16)
    w2 = w2_ref[...].astype(jnp.bfloat16)
    out = jnp.dot(hid, w2, preferred_element_type=jnp.float32)
    out_ref[...] = out + b2_ref[...]


@jax.jit
def kernel(x, w1, b1, w2, b2):
    B, S = x.shape
    H = w1.shape[1]
    A = w2.shape[1]

    TB = min(8192, B)
    nb = pl.cdiv(B, TB)

    return pl.pallas_call(
        _mlp_body,
        out_shape=jax.ShapeDtypeStruct((B, A), x.dtype),
        grid=(nb,),
        in_specs=[
            pl.BlockSpec((TB, S), lambda i: (i, 0)),
            pl.BlockSpec((S, H), lambda i: (0, 0)),
            pl.BlockSpec((1, H), lambda i: (0, 0)),
            pl.BlockSpec((H, A), lambda i: (0, 0)),
            pl.BlockSpec((1, A), lambda i: (0, 0)),
        ],
        out_specs=pl.BlockSpec((TB, A), lambda i: (i, 0)),
        compiler_params=pltpu.CompilerParams(
            dimension_semantics=("parallel",),
        ),
    )(x, w1, b1, w2, b2)
